# baseline (speedup 1.0000x reference)
import functools

import jax
import jax.numpy as jnp
from jax.experimental import pallas as pl
from jax.experimental.pallas import tpu as pltpu

_NUM_E = 1024
_DIM = 256
_EPS = 1e-12
_COMMIT = 0.25


def _vq_body(xnb_ref, wnb_ref, loss_ref, q_ref, idx_ref, *,
             nbatch, t_len, bb):
    b = pl.program_id(0)

    @pl.when(b == 0)
    def _init():
        loss_ref[...] = jnp.zeros((1, 1), jnp.float32)

    wnb = wnb_ref[...]
    part = jnp.zeros((1, 1), jnp.float32)

    for j in range(bb):
        xnb = xnb_ref[j]

        s = jax.lax.dot_general(wnb, xnb, (((1,), (0,)), ((), ())),
                                preferred_element_type=jnp.float32)
        d = 2.0 - 2.0 * s
        dmin = jnp.min(d, axis=0, keepdims=True)
        iota = jax.lax.broadcasted_iota(jnp.int32, d.shape, 0)
        idx = jnp.min(jnp.where(d == dmin, iota, _NUM_E), axis=0)
        idx_ref[j] = idx[None, :]

        onehot = (iota == idx[None, :]).astype(jnp.float32).astype(jnp.bfloat16)
        q = jax.lax.dot_general(wnb, onehot, (((0,), (0,)), ((), ())),
                                preferred_element_type=jnp.float32)
        q_ref[j] = q
        part = part + jnp.sum(dmin, keepdims=True).reshape(1, 1)

    denom = nbatch * t_len * _DIM
    loss_ref[...] += (1.0 + _COMMIT) * part / denom


def kernel(inputs, embedding_weight):
    nbatch, dim, t_len = inputs.shape
    bb = 4

    flat = jnp.transpose(inputs, (0, 2, 1)).reshape(-1, dim)
    nm = jnp.maximum(jnp.sqrt(jnp.sum(flat * flat, axis=1, keepdims=True)), _EPS)
    nmc = nm.reshape(nbatch, 1, t_len)
    xnb = (inputs / nmc).astype(jnp.bfloat16)
    w = embedding_weight
    nw = jnp.maximum(jnp.sqrt(jnp.sum(w * w, axis=1, keepdims=True)), _EPS)
    wnb = (w / nw).astype(jnp.bfloat16)

    body = functools.partial(_vq_body, nbatch=nbatch, t_len=t_len, bb=bb)
    loss2d, quantized, idx3d = pl.pallas_call(
        body,
        grid=(nbatch // bb,),
        in_specs=[
            pl.BlockSpec((bb, dim, t_len), lambda b: (b, 0, 0)),
            pl.BlockSpec((_NUM_E, dim), lambda b: (0, 0)),
        ],
        out_specs=[
            pl.BlockSpec((1, 1), lambda b: (0, 0)),
            pl.BlockSpec((bb, dim, t_len), lambda b: (b, 0, 0)),
            pl.BlockSpec((bb, 1, t_len), lambda b: (b, 0, 0)),
        ],
        out_shape=[
            jax.ShapeDtypeStruct((1, 1), jnp.float32),
            jax.ShapeDtypeStruct((nbatch, dim, t_len), jnp.float32),
            jax.ShapeDtypeStruct((nbatch, 1, t_len), jnp.int32),
        ],
    )(xnb, wnb)
    loss = loss2d[0, 0]
    encoding_indices = idx3d.reshape(nbatch * t_len, 1)
    return (loss, quantized, encoding_indices, 0)


# speedup vs baseline: 1.2235x; 1.2235x over previous
import functools

import jax
import jax.numpy as jnp
from jax.experimental import pallas as pl
from jax.experimental.pallas import tpu as pltpu

_NUM_E = 1024
_DIM = 256
_EPS = 1e-12
_COMMIT = 0.25


def _prep_body(w_ref, nw_ref, wnb_ref, wn2_ref):
    wnb = (w_ref[...] / nw_ref[...]).astype(jnp.bfloat16)
    wnb_ref[...] = wnb
    wn2_ref[...] = wnb * jnp.bfloat16(-2.0)


def _vq_body(x_ref, nm_ref, wnb_ref, wn2_ref, loss_ref, q_ref, idx_ref, *,
             nbatch, t_len, bb):
    b = pl.program_id(0)

    @pl.when(b == 0)
    def _init():
        loss_ref[...] = jnp.zeros((1, 1), jnp.float32)

    wnb = wnb_ref[...]
    wn2 = wn2_ref[...]
    part = jnp.zeros((1, 1), jnp.float32)

    shp = (_NUM_E, t_len)
    iotaf = jax.lax.broadcasted_iota(jnp.int32, shp, 0).astype(jnp.float32)
    iota = jax.lax.broadcasted_iota(jnp.int32, shp, 0)

    for j in range(bb):
        xn = x_ref[j] / nm_ref[j]
        xnb = xn.astype(jnp.bfloat16)

        s2 = jax.lax.dot_general(wn2, xnb, (((1,), (0,)), ((), ())),
                                 preferred_element_type=jnp.float32)
        d = s2 + 2.0
        dmin = jnp.min(d, axis=0, keepdims=True)
        idxf = jnp.min(jnp.where(d == dmin, iotaf, float(2 * _NUM_E)),
                       axis=0)
        idx = idxf.astype(jnp.int32)
        idx_ref[j] = idx[None, :]

        onehot = (iota == idx[None, :]).astype(jnp.float32).astype(jnp.bfloat16)
        q = jax.lax.dot_general(wnb, onehot, (((0,), (0,)), ((), ())),
                                preferred_element_type=jnp.float32)
        q_ref[j] = q
        part = part + jnp.sum(dmin, keepdims=True).reshape(1, 1)

    denom = nbatch * t_len * _DIM
    loss_ref[...] += (1.0 + _COMMIT) * part / denom


def kernel(inputs, embedding_weight):
    nbatch, dim, t_len = inputs.shape
    bb = 4

    flat = jnp.transpose(inputs, (0, 2, 1)).reshape(-1, dim)
    nm = jnp.maximum(jnp.sqrt(jnp.sum(flat * flat, axis=1, keepdims=True)), _EPS)
    nmc = nm.reshape(nbatch, 1, t_len)
    w = embedding_weight
    nw = jnp.maximum(jnp.sqrt(jnp.sum(w * w, axis=1, keepdims=True)), _EPS)

    wnb, wn2 = pl.pallas_call(
        _prep_body,
        in_specs=[pl.BlockSpec((_NUM_E, dim), lambda: (0, 0)),
                  pl.BlockSpec((_NUM_E, 1), lambda: (0, 0))],
        out_specs=[pl.BlockSpec((_NUM_E, dim), lambda: (0, 0)),
                   pl.BlockSpec((_NUM_E, dim), lambda: (0, 0))],
        out_shape=[jax.ShapeDtypeStruct((_NUM_E, dim), jnp.bfloat16),
                   jax.ShapeDtypeStruct((_NUM_E, dim), jnp.bfloat16)],
        grid=(),
    )(w, nw)

    body = functools.partial(_vq_body, nbatch=nbatch, t_len=t_len, bb=bb)
    loss2d, quantized, idx3d = pl.pallas_call(
        body,
        grid=(nbatch // bb,),
        in_specs=[
            pl.BlockSpec((bb, dim, t_len), lambda b: (b, 0, 0)),
            pl.BlockSpec((bb, 1, t_len), lambda b: (b, 0, 0)),
            pl.BlockSpec((_NUM_E, dim), lambda b: (0, 0)),
            pl.BlockSpec((_NUM_E, dim), lambda b: (0, 0)),
        ],
        out_specs=[
            pl.BlockSpec((1, 1), lambda b: (0, 0)),
            pl.BlockSpec((bb, dim, t_len), lambda b: (b, 0, 0)),
            pl.BlockSpec((bb, 1, t_len), lambda b: (b, 0, 0)),
        ],
        out_shape=[
            jax.ShapeDtypeStruct((1, 1), jnp.float32),
            jax.ShapeDtypeStruct((nbatch, dim, t_len), jnp.float32),
            jax.ShapeDtypeStruct((nbatch, 1, t_len), jnp.int32),
        ],
    )(inputs, nmc, wnb, wn2)
    loss = loss2d[0, 0]
    encoding_indices = idx3d.reshape(nbatch * t_len, 1)
    return (loss, quantized, encoding_indices, 0)
